# baseline (speedup 1.0000x reference)
import functools
from itertools import combinations

import jax
import jax.numpy as jnp
import numpy as np
from jax import lax
from jax.experimental import pallas as pl
from jax.experimental.pallas import tpu as pltpu
from jax.experimental.pallas import tpu_sc as plsc

B = 4096
F = 26
V = 100000
D = 16
H = 128
NP = 325
FD = F * D

NC = 2
NS = 16
NW = NC * NS
CHUNK = (B * F) // NW
NSTREAM = CHUNK // 128

_PAIRS = np.array(list(combinations(range(F), 2)), dtype=np.int32)

_OFFS = ((np.arange(CHUNK, dtype=np.int64) % F) * V).astype(np.int32)
_OFFS = _OFFS.reshape(NSTREAM, 128)


def _sc_gather(tab_hbm, xi_hbm, offs_hbm, out_hbm, idx_v, offs_v, rows_v, sem):
    wid = lax.axis_index("s") * NC + lax.axis_index("c")
    base = wid * CHUNK
    pltpu.sync_copy(xi_hbm.at[wid], idx_v)
    pltpu.sync_copy(offs_hbm, offs_v)
    for j in range(NSTREAM):
        for i in range(128 // 16):
            sl = pl.ds(i * 16, 16)
            idx_v[j, sl] = idx_v[j, sl] + offs_v[j, sl]
    copies = [
        pltpu.async_copy(
            tab_hbm.at[idx_v.at[j]],
            rows_v.at[pl.ds(j * 128, 128)],
            sem,
        )
        for j in range(NSTREAM)
    ]
    for cp in copies:
        cp.wait()
    pltpu.sync_copy(rows_v, out_hbm.at[pl.ds(base, CHUNK)])


@functools.partial(jax.jit, static_argnames=())
def _gather_rows(tab_flat, xi_r, offs):
    mesh = plsc.VectorSubcoreMesh(
        core_axis_name="c", subcore_axis_name="s", num_cores=NC, num_subcores=NS
    )
    return pl.kernel(
        _sc_gather,
        out_type=jax.ShapeDtypeStruct((B * F, D), jnp.float32),
        mesh=mesh,
        compiler_params=pltpu.CompilerParams(use_tc_tiling_on_sc=False),
        scratch_types=[
            pltpu.VMEM((NSTREAM, 128), jnp.int32),
            pltpu.VMEM((NSTREAM, 128), jnp.int32),
            pltpu.VMEM((CHUNK, D), jnp.float32),
            pltpu.SemaphoreType.DMA,
        ],
    )(tab_flat, xi_r, offs)


def _tc_body(eraw_ref, xvr_ref, k_ref, xv_ref, w1_ref, b1_ref, w2h_ref, b2_ref,
             out_ref):
    e = eraw_ref[...] * xvr_ref[...]
    y = jnp.dot(e, k_ref[...], preferred_element_type=jnp.float32)
    quad = 0.5 * jnp.sum(e * y, axis=1)
    h = jnp.maximum(
        jnp.dot(xv_ref[...], w1_ref[...], preferred_element_type=jnp.float32)
        + b1_ref[...], 0.0)
    dense = jnp.sum(h * w2h_ref[...], axis=1)
    out_ref[...] = (quad + dense + b2_ref[0, 0])[:, None]


def _tc_combine(eraw2, xvr, K, xv, W1, b1r, w2h, b2r):
    bm = 512
    grid = B // bm
    return pl.pallas_call(
        _tc_body,
        grid=(grid,),
        in_specs=[
            pl.BlockSpec((bm, FD), lambda i: (i, 0)),
            pl.BlockSpec((bm, FD), lambda i: (i, 0)),
            pl.BlockSpec((FD, FD), lambda i: (0, 0)),
            pl.BlockSpec((bm, F), lambda i: (i, 0)),
            pl.BlockSpec((F, H), lambda i: (0, 0)),
            pl.BlockSpec((1, H), lambda i: (0, 0)),
            pl.BlockSpec((1, H), lambda i: (0, 0)),
            pl.BlockSpec((1, 1), lambda i: (0, 0)),
        ],
        out_specs=pl.BlockSpec((bm, 1), lambda i: (i, 0)),
        out_shape=jax.ShapeDtypeStruct((B, 1), jnp.float32),
    )(eraw2, xvr, K, xv, W1, b1r, w2h, b2r)


def kernel(xv, xi, tables, W1, b1, W2, b2):
    xi32 = xi.astype(jnp.int32)
    tab_flat = tables.reshape(F * V, D)
    xi_r = xi32.reshape(NW, NSTREAM, 128)
    offs = jnp.asarray(_OFFS)

    eraw = _gather_rows(tab_flat, xi_r, offs)
    eraw2 = eraw.reshape(B, FD)

    pi = jnp.asarray(_PAIRS[:, 0])
    pj = jnp.asarray(_PAIRS[:, 1])
    w_q = W2[:NP, 0]
    A = jnp.zeros((F, F), jnp.float32).at[pi, pj].set(w_q)
    A = A + A.T
    K = jnp.einsum("fg,de->fdge", A, jnp.eye(D, dtype=jnp.float32))
    K = K.reshape(FD, FD)

    xvr = jnp.repeat(xv, D, axis=1)
    b1r = b1.reshape(1, H)
    w2h = W2[NP:, 0].reshape(1, H)
    b2r = b2.reshape(1, 1)

    return _tc_combine(eraw2, xvr, K, xv, W1, b1r, w2h, b2r)
